# Initial kernel scaffold; baseline (speedup 1.0000x reference)
import jax
import jax.numpy as jnp
from jax.experimental import pallas as pl


def kernel(x, node_rep, edge_index, batch, W_node, b_node, W_edge, b_edge):
    raise NotImplementedError("write your pallas kernel here")


# speedup vs baseline: 46.6412x; 46.6412x over previous
import jax
import jax.numpy as jnp
from jax import lax
from jax.experimental import pallas as pl
from jax.experimental.pallas import tpu as pltpu
from jax.experimental.pallas import tpu_sc as plsc

N = 10000
E = 320000
D = 128
G = 256
L = 16
NW = 32
NPAD = 10240
NODE_CHUNK = NPAD // NW
EDGE_CHUNK = E // NW
NBE = G
NBN = 272


def _dense_body(x_ref, w_ref, b_ref, nk_ref, p_ref):
    h = jnp.dot(x_ref[...], w_ref[...], preferred_element_type=jnp.float32)
    h = h + b_ref[...]
    a = h[:, 0:1]
    b = h[:, 1:2]
    m = jnp.maximum(a, b)
    ea = jnp.exp(a - m)
    eb = jnp.exp(b - m)
    nk_ref[...] = ea / (ea + eb)
    p_ref[...] = h[:, 2:6]


def _dense_call(node_rep, wcat, bcat):
    blk = 1000
    return pl.pallas_call(
        _dense_body,
        grid=(N // blk,),
        in_specs=[
            pl.BlockSpec((blk, D), lambda i: (i, 0)),
            pl.BlockSpec((D, 8), lambda i: (0, 0)),
            pl.BlockSpec((1, 8), lambda i: (0, 0)),
        ],
        out_specs=[
            pl.BlockSpec((blk, 1), lambda i: (i, 0)),
            pl.BlockSpec((blk, 4), lambda i: (i, 0)),
        ],
        out_shape=[
            jax.ShapeDtypeStruct((N, 1), jnp.float32),
            jax.ShapeDtypeStruct((N, 4), jnp.float32),
        ],
    )(node_rep, wcat, bcat)


def _sc_body(p_hbm, nk_hbm, batch_hbm, row_hbm, col_hbm, ek_hbm, out_hbm,
             p_v, batch_v, nk_v, row_v, col_v, ek_v,
             a_sn, a_cn, a_nzn, a_se, a_ce, a_nze, res_v):
    w = lax.axis_index("s") * 2 + lax.axis_index("c")

    pltpu.sync_copy(p_hbm, p_v)
    pltpu.sync_copy(batch_hbm, batch_v)
    pltpu.sync_copy(nk_hbm.at[pl.ds(w * NODE_CHUNK, NODE_CHUNK)], nk_v)
    pltpu.sync_copy(row_hbm.at[pl.ds(w * EDGE_CHUNK, EDGE_CHUNK)], row_v)
    pltpu.sync_copy(col_hbm.at[pl.ds(w * EDGE_CHUNK, EDGE_CHUNK)], col_v)

    zeros = jnp.zeros((L,), jnp.float32)
    ones = jnp.ones((L,), jnp.float32)
    lane = lax.iota(jnp.int32, L)

    def z1(i, carry):
        off = pl.ds(i * L, L)
        a_se[off] = zeros
        a_ce[off] = zeros
        a_nze[off] = zeros
        a_sn[off] = zeros
        a_cn[off] = zeros
        a_nzn[off] = zeros
        return carry

    lax.fori_loop(0, (L * NBE) // L, z1, None)

    def z2(i, carry):
        off = pl.ds(L * NBE + i * L, L)
        a_sn[off] = zeros
        a_cn[off] = zeros
        a_nzn[off] = zeros
        return carry

    lax.fori_loop(0, (L * NBN - L * NBE) // L, z2, None)

    def edge_body(i, carry):
        sl = pl.ds(i * L, L)
        r = row_v[sl] * 4
        c = col_v[sl] * 4
        p1a = plsc.load_gather(p_v, [r])
        p1b = plsc.load_gather(p_v, [r + 1])
        p2a = plsc.load_gather(p_v, [c + 2])
        p2b = plsc.load_gather(p_v, [c + 3])
        a = p1a + p2a
        b = p1b + p2b
        m = jnp.maximum(a, b)
        ea = jnp.exp(a - m)
        eb = jnp.exp(b - m)
        ek = ea / (ea + eb)
        ek_v[sl] = ek
        g = plsc.load_gather(batch_v, [row_v[sl]])
        flat = lane * NBE + g
        plsc.addupdate_scatter(a_se, [flat], ek)
        plsc.addupdate_scatter(a_ce, [flat], ones)
        plsc.addupdate_scatter(a_nze, [flat], jnp.where(ek > 0, ones, zeros))
        return carry

    lax.fori_loop(0, EDGE_CHUNK // L, edge_body, None)
    pltpu.sync_copy(ek_v, ek_hbm.at[pl.ds(w * EDGE_CHUNK, EDGE_CHUNK)])

    def node_body(i, carry):
        nk = nk_v[pl.ds(i * L, L)]
        g = batch_v[pl.ds(w * NODE_CHUNK + i * L, L)]
        flat = lane * NBN + g
        plsc.addupdate_scatter(a_sn, [flat], nk)
        plsc.addupdate_scatter(a_cn, [flat], ones)
        plsc.addupdate_scatter(a_nzn, [flat], jnp.where(nk > 0, ones, zeros))
        return carry

    lax.fori_loop(0, NODE_CHUNK // L, node_body, None)

    for k, (acc, nb) in enumerate((
            (a_sn, NBN), (a_cn, NBN), (a_nzn, NBN),
            (a_se, NBE), (a_ce, NBE), (a_nze, NBE))):
        def jbody(j, carry, acc=acc, nb=nb, k=k):
            def lbody(l, tot, acc=acc, nb=nb):
                return tot + acc[pl.ds(l * nb + j * L, L)]
            tot = lax.fori_loop(0, L, lbody, zeros)
            res_v[pl.ds(k * G + j * L, L)] = tot
            return carry
        lax.fori_loop(0, G // L, jbody, None)

    for k in range(6):
        pltpu.sync_copy(res_v.at[pl.ds(k * G, G)], out_hbm.at[k, w])


def _sc_call(p, nk_pad, batch_pad, rows, cols):
    mesh = plsc.VectorSubcoreMesh(core_axis_name="c", subcore_axis_name="s")
    kern = pl.kernel(
        _sc_body,
        mesh=mesh,
        compiler_params=pltpu.CompilerParams(needs_layout_passes=False),
        out_type=[
            jax.ShapeDtypeStruct((E,), jnp.float32),
            jax.ShapeDtypeStruct((6, NW, G), jnp.float32),
        ],
        scratch_types=[
            pltpu.VMEM((N * 4,), jnp.float32),
            pltpu.VMEM((NPAD,), jnp.int32),
            pltpu.VMEM((NODE_CHUNK,), jnp.float32),
            pltpu.VMEM((EDGE_CHUNK,), jnp.int32),
            pltpu.VMEM((EDGE_CHUNK,), jnp.int32),
            pltpu.VMEM((EDGE_CHUNK,), jnp.float32),
            pltpu.VMEM((L * NBN,), jnp.float32),
            pltpu.VMEM((L * NBN,), jnp.float32),
            pltpu.VMEM((L * NBN,), jnp.float32),
            pltpu.VMEM((L * NBE,), jnp.float32),
            pltpu.VMEM((L * NBE,), jnp.float32),
            pltpu.VMEM((L * NBE,), jnp.float32),
            pltpu.VMEM((6 * G,), jnp.float32),
        ],
    )
    return kern(p, nk_pad, batch_pad, rows, cols)


def _finish_body(x_ref, o_ref):
    x = x_ref[...]
    sn = jnp.sum(x[0:32], axis=0, keepdims=True)
    cn = jnp.sum(x[32:64], axis=0, keepdims=True)
    nzn = jnp.sum(x[64:96], axis=0, keepdims=True)
    se = jnp.sum(x[96:128], axis=0, keepdims=True)
    ce = jnp.sum(x[128:160], axis=0, keepdims=True)
    nze = jnp.sum(x[160:192], axis=0, keepdims=True)
    eps = jnp.float32(1e-8)
    o_ref[...] = jnp.concatenate(
        [sn + eps, cn - sn + eps, nzn / cn,
         se + eps, ce - se + eps, nze / ce], axis=0)


def _finish_call(partial):
    return pl.pallas_call(
        _finish_body,
        out_shape=jax.ShapeDtypeStruct((6, G), jnp.float32),
    )(partial)


def kernel(x, node_rep, edge_index, batch, W_node, b_node, W_edge, b_edge):
    wcat = jnp.zeros((D, 8), jnp.float32)
    wcat = wcat.at[:, 0:2].set(W_node)
    wcat = wcat.at[:, 2:4].set(W_edge[:D])
    wcat = wcat.at[:, 4:6].set(W_edge[D:])
    bcat = jnp.zeros((1, 8), jnp.float32)
    bcat = bcat.at[0, 0:2].set(b_node)
    bcat = bcat.at[0, 2:4].set(b_edge)

    nk, p = _dense_call(node_rep, wcat, bcat)

    nk_pad = jnp.concatenate([nk[:, 0], jnp.zeros((NPAD - N,), jnp.float32)])
    batch_pad = jnp.concatenate(
        [batch, jnp.full((NPAD - N,), G, jnp.int32)])
    ek, partial = _sc_call(p.reshape(N * 4), nk_pad, batch_pad,
                           edge_index[0], edge_index[1])

    fin = _finish_call(partial.reshape(6 * NW, G))
    return (nk, ek.reshape(E, 1),
            fin[0].reshape(G, 1), fin[1].reshape(G, 1),
            fin[3].reshape(G, 1), fin[4].reshape(G, 1),
            fin[2].reshape(G, 1), fin[5].reshape(G, 1))


# speedup vs baseline: 61.9756x; 1.3288x over previous
import jax
import jax.numpy as jnp
from jax import lax
from jax.experimental import pallas as pl
from jax.experimental.pallas import tpu as pltpu
from jax.experimental.pallas import tpu_sc as plsc

N = 10000
E = 320000
D = 128
G = 256
L = 16
NW = 32
NPAD = 10240
NODE_CHUNK = NPAD // NW
EDGE_CHUNK = E // NW
NBE = G
NBN = 272


def _dense_body(x_ref, w_ref, b_ref, nk_ref, p1a_ref, p1b_ref, p2a_ref,
                p2b_ref):
    hT = lax.dot_general(w_ref[...], x_ref[...], (((1,), (1,)), ((), ())),
                         preferred_element_type=jnp.float32)
    hT = hT + b_ref[...]
    a = hT[0:1, :]
    b = hT[1:2, :]
    m = jnp.maximum(a, b)
    ea = jnp.exp(a - m)
    eb = jnp.exp(b - m)
    blk = a.shape[1]
    nk_ref[...] = jnp.reshape(ea / (ea + eb), (blk,))
    p1a_ref[...] = jnp.reshape(hT[2:3, :], (blk,))
    p1b_ref[...] = jnp.reshape(hT[3:4, :], (blk,))
    p2a_ref[...] = jnp.reshape(hT[4:5, :], (blk,))
    p2b_ref[...] = jnp.reshape(hT[5:6, :], (blk,))


def _dense_call(node_rep, wcatT, bcat):
    out1d = jax.ShapeDtypeStruct((N,), jnp.float32)
    return pl.pallas_call(
        _dense_body,
        out_shape=[out1d] * 5,
    )(node_rep, wcatT, bcat)


def _sc_body(p1a_hbm, p1b_hbm, p2a_hbm, p2b_hbm, nk_hbm, batch_hbm,
             row_hbm, col_hbm, ek_hbm, out_hbm,
             p1a_v, p1b_v, p2a_v, p2b_v, batch_v, nk_v, row_v, col_v, ek_v,
             a_sn, a_cn, a_nzn, a_se, a_ce, a_nze, res_v):
    w = lax.axis_index("s") * 2 + lax.axis_index("c")

    pltpu.sync_copy(p1a_hbm, p1a_v)
    pltpu.sync_copy(p1b_hbm, p1b_v)
    pltpu.sync_copy(p2a_hbm, p2a_v)
    pltpu.sync_copy(p2b_hbm, p2b_v)
    pltpu.sync_copy(batch_hbm, batch_v)
    pltpu.sync_copy(nk_hbm.at[pl.ds(w * NODE_CHUNK, NODE_CHUNK)], nk_v)
    pltpu.sync_copy(row_hbm.at[pl.ds(w * EDGE_CHUNK, EDGE_CHUNK)], row_v)
    pltpu.sync_copy(col_hbm.at[pl.ds(w * EDGE_CHUNK, EDGE_CHUNK)], col_v)

    zeros = jnp.zeros((L,), jnp.float32)
    ones = jnp.ones((L,), jnp.float32)
    lane = lax.iota(jnp.int32, L)

    def z1(i, carry):
        off = pl.ds(i * L, L)
        a_se[off] = zeros
        a_ce[off] = zeros
        a_nze[off] = zeros
        a_sn[off] = zeros
        a_cn[off] = zeros
        a_nzn[off] = zeros
        return carry

    lax.fori_loop(0, (L * NBE) // L, z1, None)

    def z2(i, carry):
        off = pl.ds(L * NBE + i * L, L)
        a_sn[off] = zeros
        a_cn[off] = zeros
        a_nzn[off] = zeros
        return carry

    lax.fori_loop(0, (L * NBN - L * NBE) // L, z2, None)

    def edge_body(i, carry):
        sl = pl.ds(i * L, L)
        r = row_v[sl]
        c = col_v[sl]
        p1a = plsc.load_gather(p1a_v, [r])
        p1b = plsc.load_gather(p1b_v, [r])
        p2a = plsc.load_gather(p2a_v, [c])
        p2b = plsc.load_gather(p2b_v, [c])
        a = p1a + p2a
        b = p1b + p2b
        m = jnp.maximum(a, b)
        ea = jnp.exp(a - m)
        eb = jnp.exp(b - m)
        ek = ea / (ea + eb)
        ek_v[sl] = ek
        g = plsc.load_gather(batch_v, [r])
        flat = lane * NBE + g
        plsc.addupdate_scatter(a_se, [flat], ek)
        plsc.addupdate_scatter(a_ce, [flat], ones)
        plsc.addupdate_scatter(a_nze, [flat], jnp.where(ek > 0, ones, zeros))
        return carry

    lax.fori_loop(0, EDGE_CHUNK // L, edge_body, None)
    pltpu.sync_copy(ek_v, ek_hbm.at[pl.ds(w * EDGE_CHUNK, EDGE_CHUNK)])

    def node_body(i, carry):
        nk = nk_v[pl.ds(i * L, L)]
        g = batch_v[pl.ds(w * NODE_CHUNK + i * L, L)]
        flat = lane * NBN + g
        plsc.addupdate_scatter(a_sn, [flat], nk)
        plsc.addupdate_scatter(a_cn, [flat], ones)
        plsc.addupdate_scatter(a_nzn, [flat], jnp.where(nk > 0, ones, zeros))
        return carry

    lax.fori_loop(0, NODE_CHUNK // L, node_body, None)

    for k, (acc, nb) in enumerate((
            (a_sn, NBN), (a_cn, NBN), (a_nzn, NBN),
            (a_se, NBE), (a_ce, NBE), (a_nze, NBE))):
        def jbody(j, carry, acc=acc, nb=nb, k=k):
            def lbody(l, tot, acc=acc, nb=nb):
                return tot + acc[pl.ds(l * nb + j * L, L)]
            tot = lax.fori_loop(0, L, lbody, zeros)
            res_v[pl.ds(k * G + j * L, L)] = tot
            return carry
        lax.fori_loop(0, G // L, jbody, None)

    for k in range(6):
        pltpu.sync_copy(res_v.at[pl.ds(k * G, G)], out_hbm.at[k, w])


def _sc_call(p1a, p1b, p2a, p2b, nk_pad, batch_pad, rows, cols):
    mesh = plsc.VectorSubcoreMesh(core_axis_name="c", subcore_axis_name="s")
    kern = pl.kernel(
        _sc_body,
        mesh=mesh,
        compiler_params=pltpu.CompilerParams(needs_layout_passes=False),
        out_type=[
            jax.ShapeDtypeStruct((E,), jnp.float32),
            jax.ShapeDtypeStruct((6, NW, G), jnp.float32),
        ],
        scratch_types=[
            pltpu.VMEM((N,), jnp.float32),
            pltpu.VMEM((N,), jnp.float32),
            pltpu.VMEM((N,), jnp.float32),
            pltpu.VMEM((N,), jnp.float32),
            pltpu.VMEM((NPAD,), jnp.int32),
            pltpu.VMEM((NODE_CHUNK,), jnp.float32),
            pltpu.VMEM((EDGE_CHUNK,), jnp.int32),
            pltpu.VMEM((EDGE_CHUNK,), jnp.int32),
            pltpu.VMEM((EDGE_CHUNK,), jnp.float32),
            pltpu.VMEM((L * NBN,), jnp.float32),
            pltpu.VMEM((L * NBN,), jnp.float32),
            pltpu.VMEM((L * NBN,), jnp.float32),
            pltpu.VMEM((L * NBE,), jnp.float32),
            pltpu.VMEM((L * NBE,), jnp.float32),
            pltpu.VMEM((L * NBE,), jnp.float32),
            pltpu.VMEM((6 * G,), jnp.float32),
        ],
    )
    return kern(p1a, p1b, p2a, p2b, nk_pad, batch_pad, rows, cols)


def _finish_body(x_ref, o_ref):
    x = x_ref[...]
    sn = jnp.sum(x[0:32], axis=0, keepdims=True)
    cn = jnp.sum(x[32:64], axis=0, keepdims=True)
    nzn = jnp.sum(x[64:96], axis=0, keepdims=True)
    se = jnp.sum(x[96:128], axis=0, keepdims=True)
    ce = jnp.sum(x[128:160], axis=0, keepdims=True)
    nze = jnp.sum(x[160:192], axis=0, keepdims=True)
    eps = jnp.float32(1e-8)
    o_ref[...] = jnp.concatenate(
        [sn + eps, cn - sn + eps, nzn / cn,
         se + eps, ce - se + eps, nze / ce], axis=0)


def _finish_call(partial):
    return pl.pallas_call(
        _finish_body,
        out_shape=jax.ShapeDtypeStruct((6, G), jnp.float32),
    )(partial)


def kernel(x, node_rep, edge_index, batch, W_node, b_node, W_edge, b_edge):
    wcatT = jnp.zeros((8, D), jnp.float32)
    wcatT = wcatT.at[0:2].set(W_node.T)
    wcatT = wcatT.at[2:4].set(W_edge[:D].T)
    wcatT = wcatT.at[4:6].set(W_edge[D:].T)
    bcat = jnp.concatenate(
        [b_node, b_edge, jnp.zeros((4,), jnp.float32)]).reshape(8, 1)

    nk1, p1a, p1b, p2a, p2b = _dense_call(node_rep, wcatT, bcat)

    nk_pad = jnp.concatenate([nk1, jnp.zeros((NPAD - N,), jnp.float32)])
    batch_pad = jnp.concatenate(
        [batch, jnp.full((NPAD - N,), G, jnp.int32)])
    ek, partial = _sc_call(p1a, p1b, p2a, p2b, nk_pad, batch_pad,
                           edge_index[0], edge_index[1])

    fin = _finish_call(partial.reshape(6 * NW, G))
    return (nk1.reshape(N, 1), ek.reshape(E, 1),
            fin[0].reshape(G, 1), fin[1].reshape(G, 1),
            fin[3].reshape(G, 1), fin[4].reshape(G, 1),
            fin[2].reshape(G, 1), fin[5].reshape(G, 1))


# speedup vs baseline: 137.3916x; 2.2169x over previous
import jax
import jax.numpy as jnp
from jax import lax
from jax.experimental import pallas as pl
from jax.experimental.pallas import tpu as pltpu
from jax.experimental.pallas import tpu_sc as plsc

N = 10000
E = 320000
D = 128
G = 256
L = 16
NW = 32
EC0 = 10240
ECL = E - 31 * EC0
NC0 = 320
NCL = N - 31 * NC0
NB = G


def _pack_bf16_pair(hi, lo):
    h16 = lax.bitcast_convert_type(hi.astype(jnp.bfloat16), jnp.uint16)
    l16 = lax.bitcast_convert_type(lo.astype(jnp.bfloat16), jnp.uint16)
    word = (h16.astype(jnp.uint32) << 16) | l16.astype(jnp.uint32)
    return jnp.reshape(lax.bitcast_convert_type(word, jnp.int32),
                       (hi.shape[1],))


def _dense_body(x_ref, wn_ref, we_ref, b_ref, nk_ref, e1_ref, e2_ref):
    x = x_ref[...]
    dn = (((0,), (1,)), ((), ()))
    hn = lax.dot_general(wn_ref[...], x, dn,
                         preferred_element_type=jnp.float32)
    hn = hn + b_ref[0:2, :]
    a = hn[0:1, :]
    b = hn[1:2, :]
    m = jnp.maximum(a, b)
    ea = jnp.exp(a - m)
    eb = jnp.exp(b - m)
    nk_ref[...] = jnp.reshape(ea / (ea + eb), (x.shape[0],))
    h1 = lax.dot_general(we_ref[0:D], x, dn,
                         preferred_element_type=jnp.float32)
    h1 = h1 + b_ref[2:4, :]
    h2 = lax.dot_general(we_ref[D:2 * D], x, dn,
                         preferred_element_type=jnp.float32)
    e1_ref[...] = _pack_bf16_pair(h1[0:1, :], h1[1:2, :])
    e2_ref[...] = _pack_bf16_pair(h2[0:1, :], h2[1:2, :])


def _dense_call(node_rep, W_node, W_edge, bcat):
    return pl.pallas_call(
        _dense_body,
        out_shape=[
            jax.ShapeDtypeStruct((N,), jnp.float32),
            jax.ShapeDtypeStruct((N,), jnp.int32),
            jax.ShapeDtypeStruct((N,), jnp.int32),
        ],
    )(node_rep, W_node, W_edge, bcat)


def _sc_body(e1_hbm, e2_hbm, nk_hbm, batch_hbm, ei_hbm,
             ek_hbm, out_hbm,
             e1_v, e2_v, batch_v, nk_v, ei_v, ek_v,
             a_sn, a_cn, a_se, a_ce, res_v, dma_sem):
    w = lax.axis_index("s") * 2 + lax.axis_index("c")
    is_last = w == NW - 1

    copies = [
        pltpu.async_copy(e1_hbm, e1_v, dma_sem),
        pltpu.async_copy(e2_hbm, e2_v, dma_sem),
        pltpu.async_copy(batch_hbm, batch_v, dma_sem),
    ]

    @pl.when(jnp.logical_not(is_last))
    def _():
        pltpu.sync_copy(ei_hbm.at[:, pl.ds(w * EC0, EC0)], ei_v)

    @pl.when(is_last)
    def _():
        pltpu.sync_copy(ei_hbm.at[:, pl.ds(31 * EC0, ECL)],
                        ei_v.at[:, pl.ds(0, ECL)])

    zeros = jnp.zeros((L,), jnp.float32)
    ones = jnp.ones((L,), jnp.float32)
    lane = lax.iota(jnp.int32, L)
    mask_hi = jnp.full((L,), -65536, jnp.int32)

    izeros = jnp.zeros((L,), jnp.int32)

    @plsc.parallel_loop(0, NB, unroll=8)
    def z1(i):
        off = pl.ds(i * L, L)
        a_se[off] = zeros
        a_ce[off] = izeros
        a_sn[off] = zeros
        a_cn[off] = izeros

    for cp in copies:
        cp.wait()

    nv_edge = jnp.where(is_last, ECL // L, EC0 // L)

    @plsc.parallel_loop(0, nv_edge, unroll=8)
    def edge_body(i):
        sl = pl.ds(i * L, L)
        r = ei_v[0, sl]
        c = ei_v[1, sl]
        w1 = plsc.load_gather(e1_v, [r])
        w2 = plsc.load_gather(e2_v, [c])
        p1a = plsc.bitcast(w1 & mask_hi, jnp.float32)
        p1b = plsc.bitcast(w1 << 16, jnp.float32)
        p2a = plsc.bitcast(w2 & mask_hi, jnp.float32)
        p2b = plsc.bitcast(w2 << 16, jnp.float32)
        a = p1a + p2a
        b = p1b + p2b
        m = jnp.maximum(a, b)
        ea = jnp.exp(a - m)
        eb = jnp.exp(b - m)
        ek = ea / (ea + eb)
        ek_v[sl] = ek
        g = plsc.load_gather(batch_v, [r])
        flat = lane * NB + g
        plsc.addupdate_scatter(a_se, [flat], ek)
        cnz = jnp.where(ek > 0, jnp.full((L,), 0x10001, jnp.int32),
                        jnp.full((L,), 0x10000, jnp.int32))
        plsc.addupdate_scatter(a_ce, [flat], cnz)

    @pl.when(jnp.logical_not(is_last))
    def _():
        pltpu.sync_copy(ek_v, ek_hbm.at[0, pl.ds(w * EC0, EC0)])

    @pl.when(is_last)
    def _():
        pltpu.sync_copy(ek_v.at[pl.ds(0, ECL)],
                        ek_hbm.at[0, pl.ds(31 * EC0, ECL)])

    @pl.when(jnp.logical_not(is_last))
    def _():
        pltpu.sync_copy(nk_hbm.at[pl.ds(w * NC0, NC0)], nk_v)

    @pl.when(is_last)
    def _():
        pltpu.sync_copy(nk_hbm.at[pl.ds(31 * NC0, NCL)],
                        nk_v.at[pl.ds(0, NCL)])

    nv_node = jnp.where(is_last, NCL // L, NC0 // L)

    @plsc.parallel_loop(0, nv_node, unroll=1)
    def node_body(i):
        nk = nk_v[pl.ds(i * L, L)]
        g = batch_v[pl.ds(w * NC0 + i * L, L)]
        flat = lane * NB + g
        plsc.addupdate_scatter(a_sn, [flat], nk)
        cnz = jnp.where(nk > 0, jnp.full((L,), 0x10001, jnp.int32),
                        jnp.full((L,), 0x10000, jnp.int32))
        plsc.addupdate_scatter(a_cn, [flat], cnz)

    mask_lo = jnp.full((L,), 0xFFFF, jnp.int32)
    for k, acc, packed in ((0, a_sn, False), (1, a_cn, True),
                           (3, a_se, False), (4, a_ce, True)):
        @plsc.parallel_loop(0, G // L, unroll=2)
        def jbody(j, acc=acc, k=k, packed=packed):
            vals = [acc[pl.ds(l * NB + j * L, L)] for l in range(L)]
            while len(vals) > 1:
                vals = [a + b for a, b in zip(vals[::2], vals[1::2])]
            tot = vals[0]
            if packed:
                res_v[pl.ds(k * G + j * L, L)] = (
                    lax.shift_right_logical(tot, 16).astype(jnp.float32))
                res_v[pl.ds((k + 1) * G + j * L, L)] = (
                    (tot & mask_lo).astype(jnp.float32))
            else:
                res_v[pl.ds(k * G + j * L, L)] = tot

    pltpu.sync_copy(res_v, out_hbm.at[w])


def _sc_call(e1, e2, nk, batch, edge_index):
    mesh = plsc.VectorSubcoreMesh(core_axis_name="c", subcore_axis_name="s")
    kern = pl.kernel(
        _sc_body,
        mesh=mesh,
        compiler_params=pltpu.CompilerParams(needs_layout_passes=False),
        out_type=[
            jax.ShapeDtypeStruct((1, E), jnp.float32),
            jax.ShapeDtypeStruct((NW, 6 * G), jnp.float32),
        ],
        scratch_types=[
            pltpu.VMEM((N,), jnp.int32),
            pltpu.VMEM((N,), jnp.int32),
            pltpu.VMEM((N,), jnp.int32),
            pltpu.VMEM((NC0,), jnp.float32),
            pltpu.VMEM((2, EC0), jnp.int32),
            pltpu.VMEM((EC0,), jnp.float32),
            pltpu.VMEM((L * NB,), jnp.float32),
            pltpu.VMEM((L * NB,), jnp.int32),
            pltpu.VMEM((L * NB,), jnp.float32),
            pltpu.VMEM((L * NB,), jnp.int32),
            pltpu.VMEM((6 * G,), jnp.float32),
            pltpu.SemaphoreType.DMA,
        ],
    )
    return kern(e1, e2, nk, batch, edge_index)


def _finish_body(x_ref, kn_ref, en_ref, ken_ref, een_ref, nzn_ref, nze_ref):
    tot = jnp.sum(x_ref[...], axis=0)
    sn = jnp.reshape(tot[0 * G:1 * G], (1, G))
    cn = jnp.reshape(tot[1 * G:2 * G], (1, G))
    nzn = jnp.reshape(tot[2 * G:3 * G], (1, G))
    se = jnp.reshape(tot[3 * G:4 * G], (1, G))
    ce = jnp.reshape(tot[4 * G:5 * G], (1, G))
    nze = jnp.reshape(tot[5 * G:6 * G], (1, G))
    eps = jnp.float32(1e-8)
    kn_ref[...] = sn + eps
    en_ref[...] = cn - sn + eps
    ken_ref[...] = se + eps
    een_ref[...] = ce - se + eps
    nzn_ref[...] = nzn / cn
    nze_ref[...] = nze / ce


def _finish_call(partial):
    return pl.pallas_call(
        _finish_body,
        out_shape=[jax.ShapeDtypeStruct((1, G), jnp.float32)] * 6,
    )(partial)


def kernel(x, node_rep, edge_index, batch, W_node, b_node, W_edge, b_edge):
    bcat = jnp.concatenate(
        [b_node, b_edge, jnp.zeros((4,), jnp.float32)]).reshape(8, 1)

    nk1, e1, e2 = _dense_call(node_rep, W_node, W_edge, bcat)

    ek, partial = _sc_call(e1, e2, nk1, batch, edge_index)

    kn, en, ken, een, nzn, nze = _finish_call(partial)
    return (nk1.reshape(N, 1), ek.reshape(E, 1),
            kn.reshape(G, 1), en.reshape(G, 1),
            ken.reshape(G, 1), een.reshape(G, 1),
            nzn.reshape(G, 1), nze.reshape(G, 1))
